# Initial kernel scaffold; baseline (speedup 1.0000x reference)
import jax
import jax.numpy as jnp
from jax.experimental import pallas as pl


def kernel(x, edge_index, Wl0, Wr0, b0, Wl1, Wr1, b1, Wl2, Wr2, b2):
    raise NotImplementedError("write your pallas kernel here")


# speedup vs baseline: 2.7100x; 2.7100x over previous
import functools

import jax
import jax.numpy as jnp
from jax import lax
from jax.experimental import pallas as pl
from jax.experimental.pallas import tpu as pltpu
from jax.experimental.pallas import tpu_sc as plsc

N = 10000
D = 128
E = 320000
NC, NS = 2, 16
NW = NC * NS
C = 128
NCHK = 80
EPW = NCHK * C
EP = NW * EPW
NPAIR = NCHK // 2
NPAD = 10080
CR = 80
NZCH = NPAD // CR
NOCH = N // CR
MAXZ = -(-NZCH // NS)
MAXO = -(-NOCH // NS)

_MESH = plsc.VectorSubcoreMesh(
    core_axis_name="c", subcore_axis_name="s", num_cores=NC, num_subcores=NS)


def _sc_agg_body(h_hbm, src_hbm, dst_hbm, psum, sidx, didx, rows0, rows1,
                 zrow, agg_sh, gsem0, gsem1):
    core = lax.axis_index("c")
    sub = lax.axis_index("s")
    wid = sub * NC + core
    row0 = wid * NCHK

    def _zero_zrow(i, _):
        for j in range(D // 16):
            zrow[i, pl.ds(j * 16, 16)] = jnp.zeros((16,), jnp.float32)
        return 0
    lax.fori_loop(0, CR, _zero_zrow, 0)
    for k in range(MAXZ):
        ch = k * NS + sub
        @pl.when(ch < NZCH)
        def _():
            pltpu.sync_copy(zrow, agg_sh.at[pl.ds(ch * CR, CR)])

    pltpu.sync_copy(src_hbm.at[pl.ds(row0, NCHK)], sidx)
    pltpu.sync_copy(dst_hbm.at[pl.ds(row0, NCHK)], didx)

    plsc.subcore_barrier()

    def _chunkg(ch, _):
        pltpu.async_copy(h_hbm.at[sidx.at[ch]], rows0, gsem0).wait()
        pltpu.sync_copy(rows0, agg_sh.at[didx.at[ch]], add=True)
        return 0
    lax.fori_loop(0, NCHK, _chunkg, 0)

    plsc.subcore_barrier()

    for k in range(MAXO):
        ch = k * NS + sub
        @pl.when(ch < NOCH)
        def _():
            pltpu.sync_copy(agg_sh.at[pl.ds(ch * CR, CR)],
                            psum.at[pl.ds(core * N + ch * CR, CR)])


_sc_agg = pl.kernel(
    _sc_agg_body,
    out_type=[jax.ShapeDtypeStruct((2 * N, D), jnp.float32)],
    mesh=_MESH,
    scratch_types=[
        pltpu.VMEM((NCHK, C), jnp.int32),
        pltpu.VMEM((NCHK, C), jnp.int32),
        pltpu.VMEM((C, D), jnp.float32),
        pltpu.VMEM((C, D), jnp.float32),
        pltpu.VMEM((CR, D), jnp.float32),
        pltpu.VMEM_SHARED((NPAD, D), jnp.float32),
        pltpu.SemaphoreType.DMA,
        pltpu.SemaphoreType.DMA,
    ],
)


def _sc_cnt_body(dst_hbm, pcnt, didx, ones, cnt_sh):
    core = lax.axis_index("c")
    sub = lax.axis_index("s")
    wid = sub * NC + core

    def _fill_ones(i, _):
        for j in range(D // 16):
            ones[i, pl.ds(j * 16, 16)] = jnp.ones((16,), jnp.float32)
        return 0
    lax.fori_loop(0, C, _fill_ones, 0)
    def _zero_ones(i, _):
        for j in range(D // 16):
            ones[i, pl.ds(j * 16, 16)] = jnp.zeros((16,), jnp.float32)
        return 0
    lax.fori_loop(0, CR, _zero_ones, 0)
    for k in range(MAXZ):
        ch = k * NS + sub
        @pl.when(ch < NZCH)
        def _():
            pltpu.sync_copy(ones.at[pl.ds(0, CR)], cnt_sh.at[pl.ds(ch * CR, CR)])
    lax.fori_loop(0, C, _fill_ones, 0)

    pltpu.sync_copy(dst_hbm.at[pl.ds(wid * NCHK, NCHK)], didx)

    plsc.subcore_barrier()

    def _chunk(ch, _):
        pltpu.sync_copy(ones, cnt_sh.at[didx.at[ch]], add=True)
        return 0
    lax.fori_loop(0, NCHK, _chunk, 0)

    plsc.subcore_barrier()

    for k in range(MAXO):
        ch = k * NS + sub
        @pl.when(ch < NOCH)
        def _():
            pltpu.sync_copy(cnt_sh.at[pl.ds(ch * CR, CR)],
                            pcnt.at[pl.ds(core * N + ch * CR, CR)])


_sc_cnt = pl.kernel(
    _sc_cnt_body,
    out_type=[jax.ShapeDtypeStruct((2 * N, D), jnp.float32)],
    mesh=_MESH,
    scratch_types=[
        pltpu.VMEM((NCHK, C), jnp.int32),
        pltpu.VMEM((C, D), jnp.float32),
        pltpu.VMEM_SHARED((NPAD, D), jnp.float32),
    ],
)


def _tc_layer0_body(hb, p0, p1, c0, c1, wl, wr, bb, ob, invb):
    cnt = c0[:, 0:1] + c1[:, 0:1]
    inv = 1.0 / jnp.maximum(cnt, 1.0)
    agg = (p0[...] + p1[...]) * inv
    out = (jnp.dot(agg, wl[...], preferred_element_type=jnp.float32,
                   precision=lax.Precision.HIGHEST)
           + jnp.dot(hb[...], wr[...], preferred_element_type=jnp.float32,
                     precision=lax.Precision.HIGHEST)
           + bb[...])
    ob[...] = jnp.maximum(out, 0.0)
    invb[...] = inv


def _tc_layer_body(relu, hb, p0, p1, invb, wl, wr, bb, ob):
    agg = (p0[...] + p1[...]) * invb[...]
    out = (jnp.dot(agg, wl[...], preferred_element_type=jnp.float32,
                   precision=lax.Precision.HIGHEST)
           + jnp.dot(hb[...], wr[...], preferred_element_type=jnp.float32,
                     precision=lax.Precision.HIGHEST)
           + bb[...])
    ob[...] = jnp.maximum(out, 0.0) if relu else out


_R = 1000


def _tc_layer0(h, psum, pcnt, Wl, Wr, b):
    g = N // _R
    return pl.pallas_call(
        _tc_layer0_body,
        grid=(g,),
        in_specs=[
            pl.BlockSpec((_R, D), lambda i: (i, 0)),
            pl.BlockSpec((_R, D), lambda i: (i, 0)),
            pl.BlockSpec((_R, D), lambda i: (i + g, 0)),
            pl.BlockSpec((_R, D), lambda i: (i, 0)),
            pl.BlockSpec((_R, D), lambda i: (i + g, 0)),
            pl.BlockSpec((D, D), lambda i: (0, 0)),
            pl.BlockSpec((D, D), lambda i: (0, 0)),
            pl.BlockSpec((1, D), lambda i: (0, 0)),
        ],
        out_specs=[
            pl.BlockSpec((_R, D), lambda i: (i, 0)),
            pl.BlockSpec((_R, 1), lambda i: (i, 0)),
        ],
        out_shape=[
            jax.ShapeDtypeStruct((N, D), jnp.float32),
            jax.ShapeDtypeStruct((N, 1), jnp.float32),
        ],
    )(h, psum, psum, pcnt, pcnt, Wl, Wr, b.reshape(1, D))


def _tc_layer(h, psum, inv, Wl, Wr, b, relu):
    g = N // _R
    return pl.pallas_call(
        functools.partial(_tc_layer_body, relu),
        grid=(g,),
        in_specs=[
            pl.BlockSpec((_R, D), lambda i: (i, 0)),
            pl.BlockSpec((_R, D), lambda i: (i, 0)),
            pl.BlockSpec((_R, D), lambda i: (i + g, 0)),
            pl.BlockSpec((_R, 1), lambda i: (i, 0)),
            pl.BlockSpec((D, D), lambda i: (0, 0)),
            pl.BlockSpec((D, D), lambda i: (0, 0)),
            pl.BlockSpec((1, D), lambda i: (0, 0)),
        ],
        out_specs=pl.BlockSpec((_R, D), lambda i: (i, 0)),
        out_shape=jax.ShapeDtypeStruct((N, D), jnp.float32),
    )(h, psum, psum, inv, Wl, Wr, b.reshape(1, D))


def kernel(x, edge_index, Wl0, Wr0, b0, Wl1, Wr1, b1, Wl2, Wr2, b2):
    src = edge_index[0].astype(jnp.int32)
    dst = edge_index[1].astype(jnp.int32)
    pad = EP - E
    src3 = jnp.concatenate(
        [src, jnp.zeros((pad,), jnp.int32)]).reshape(NW * NCHK, C)
    dst3 = jnp.concatenate(
        [dst, N + (jnp.arange(pad, dtype=jnp.int32) % (NPAD - N))]
    ).reshape(NW * NCHK, C)

    (pcnt,) = _sc_cnt(dst3)
    (psum0,) = _sc_agg(x, src3, dst3)
    h1, inv = _tc_layer0(x, psum0, pcnt, Wl0, Wr0, b0)
    (psum1,) = _sc_agg(h1, src3, dst3)
    h2 = _tc_layer(h1, psum1, inv, Wl1, Wr1, b1, True)
    (psum2,) = _sc_agg(h2, src3, dst3)
    return _tc_layer(h2, psum2, inv, Wl2, Wr2, b2, False)


# speedup vs baseline: 3.5927x; 1.3257x over previous
import functools

import jax
import jax.numpy as jnp
from jax import lax
from jax.experimental import pallas as pl
from jax.experimental.pallas import tpu as pltpu
from jax.experimental.pallas import tpu_sc as plsc

N = 10000
D = 128
E = 320000
NC, NS = 2, 16
NW = NC * NS
C = 128
NCHK0 = 120
NCHK1 = 40
NCHKT = NCHK0 + NCHK1
EP = NS * NCHKT * C
XROW = NS * NCHKT
BW = 8
NPAD = 10080
CR = 80
NZCH = NPAD // CR
NOCH = N // CR
MAXZ = -(-NZCH // NS)
MAXO = -(-NOCH // NS)

_MESH = plsc.VectorSubcoreMesh(
    core_axis_name="c", subcore_axis_name="s", num_cores=NC, num_subcores=NS)


def _sc_agg_body(h_hbm, src_hbm, dst_hbm, psum, sidx, didx, rows0, rows1,
                 zrow, agg_sh, gsem0, gsem1):
    core = lax.axis_index("c")
    sub = lax.axis_index("s")
    row0 = sub * NCHKT + core * NCHK0

    def _zero_zrow(i, _):
        for j in range(D // 16):
            zrow[i, pl.ds(j * 16, 16)] = jnp.zeros((16,), jnp.float32)
        return 0
    lax.fori_loop(0, CR, _zero_zrow, 0)
    for k in range(MAXZ):
        ch = k * NS + sub
        @pl.when(ch < NZCH)
        def _():
            pltpu.sync_copy(zrow, agg_sh.at[pl.ds(ch * CR, CR)])

    nblk = jnp.where(core == 0, NCHK0 // BW, NCHK1 // BW)

    plsc.subcore_barrier()

    def _block(b, _):
        r = row0 + b * BW
        pltpu.sync_copy(src_hbm.at[pl.ds(r, BW)], sidx)
        pltpu.sync_copy(dst_hbm.at[pl.ds(r, BW)], didx)
        bufs = (rows0, rows1)
        sems = (gsem0, gsem1)
        pltpu.async_copy(h_hbm.at[sidx.at[0]], rows0, gsem0)
        for j in range(BW):
            rj, sj = bufs[j % 2], sems[j % 2]
            pltpu.make_async_copy(h_hbm.at[sidx.at[j]], rj, sj).wait()
            if j + 1 < BW:
                pltpu.async_copy(
                    h_hbm.at[sidx.at[j + 1]], bufs[(j + 1) % 2],
                    sems[(j + 1) % 2])
            pltpu.sync_copy(rj, agg_sh.at[didx.at[j]], add=True)
        return 0
    lax.fori_loop(0, nblk, _block, 0)

    plsc.subcore_barrier()

    for k in range(MAXO):
        ch = k * NS + sub
        @pl.when(ch < NOCH)
        def _():
            pltpu.sync_copy(agg_sh.at[pl.ds(ch * CR, CR)],
                            psum.at[pl.ds(core * N + ch * CR, CR)])


_sc_agg = pl.kernel(
    _sc_agg_body,
    out_type=[jax.ShapeDtypeStruct((2 * N, D), jnp.float32)],
    mesh=_MESH,
    scratch_types=[
        pltpu.VMEM((BW, C), jnp.int32),
        pltpu.VMEM((BW, C), jnp.int32),
        pltpu.VMEM((C, D), jnp.float32),
        pltpu.VMEM((C, D), jnp.float32),
        pltpu.VMEM((CR, D), jnp.float32),
        pltpu.VMEM_SHARED((NPAD, D), jnp.float32),
        pltpu.SemaphoreType.DMA,
        pltpu.SemaphoreType.DMA,
    ],
)


def _sc_cnt_body(dst_hbm, pcnt, didx, ones, cnt_sh):
    core = lax.axis_index("c")
    sub = lax.axis_index("s")
    row0 = sub * NCHKT + core * NCHK0
    nblk = jnp.where(core == 0, NCHK0 // BW, NCHK1 // BW)

    def _fill_ones(i, _):
        for j in range(D // 16):
            ones[i, pl.ds(j * 16, 16)] = jnp.ones((16,), jnp.float32)
        return 0
    lax.fori_loop(0, C, _fill_ones, 0)
    def _zero_ones(i, _):
        for j in range(D // 16):
            ones[i, pl.ds(j * 16, 16)] = jnp.zeros((16,), jnp.float32)
        return 0
    lax.fori_loop(0, CR, _zero_ones, 0)
    for k in range(MAXZ):
        ch = k * NS + sub
        @pl.when(ch < NZCH)
        def _():
            pltpu.sync_copy(ones.at[pl.ds(0, CR)], cnt_sh.at[pl.ds(ch * CR, CR)])
    lax.fori_loop(0, C, _fill_ones, 0)

    plsc.subcore_barrier()

    def _block(b, _):
        pltpu.sync_copy(dst_hbm.at[pl.ds(row0 + b * BW, BW)], didx)
        for j in range(BW):
            pltpu.sync_copy(ones, cnt_sh.at[didx.at[j]], add=True)
        return 0
    lax.fori_loop(0, nblk, _block, 0)

    plsc.subcore_barrier()

    for k in range(MAXO):
        ch = k * NS + sub
        @pl.when(ch < NOCH)
        def _():
            pltpu.sync_copy(cnt_sh.at[pl.ds(ch * CR, CR)],
                            pcnt.at[pl.ds(core * N + ch * CR, CR)])


_sc_cnt = pl.kernel(
    _sc_cnt_body,
    out_type=[jax.ShapeDtypeStruct((2 * N, D), jnp.float32)],
    mesh=_MESH,
    scratch_types=[
        pltpu.VMEM((BW, C), jnp.int32),
        pltpu.VMEM((C, D), jnp.float32),
        pltpu.VMEM_SHARED((NPAD, D), jnp.float32),
    ],
)


def _tc_layer0_body(hb, p0, p1, c0, c1, wl, wr, bb, ob, invb):
    cnt = c0[:, 0:1] + c1[:, 0:1]
    inv = 1.0 / jnp.maximum(cnt, 1.0)
    agg = (p0[...] + p1[...]) * inv
    out = (jnp.dot(agg, wl[...], preferred_element_type=jnp.float32,
                   precision=lax.Precision.HIGHEST)
           + jnp.dot(hb[...], wr[...], preferred_element_type=jnp.float32,
                     precision=lax.Precision.HIGHEST)
           + bb[...])
    ob[...] = jnp.maximum(out, 0.0)
    invb[...] = inv


def _tc_layer_body(relu, hb, p0, p1, invb, wl, wr, bb, ob):
    agg = (p0[...] + p1[...]) * invb[...]
    out = (jnp.dot(agg, wl[...], preferred_element_type=jnp.float32,
                   precision=lax.Precision.HIGHEST)
           + jnp.dot(hb[...], wr[...], preferred_element_type=jnp.float32,
                     precision=lax.Precision.HIGHEST)
           + bb[...])
    ob[...] = jnp.maximum(out, 0.0) if relu else out


_R = 1000


def _tc_layer0(h, psum, pcnt, Wl, Wr, b):
    g = N // _R
    return pl.pallas_call(
        _tc_layer0_body,
        grid=(g,),
        in_specs=[
            pl.BlockSpec((_R, D), lambda i: (i, 0)),
            pl.BlockSpec((_R, D), lambda i: (i, 0)),
            pl.BlockSpec((_R, D), lambda i: (i + g, 0)),
            pl.BlockSpec((_R, D), lambda i: (i, 0)),
            pl.BlockSpec((_R, D), lambda i: (i + g, 0)),
            pl.BlockSpec((D, D), lambda i: (0, 0)),
            pl.BlockSpec((D, D), lambda i: (0, 0)),
            pl.BlockSpec((1, D), lambda i: (0, 0)),
        ],
        out_specs=[
            pl.BlockSpec((_R, D), lambda i: (i, 0)),
            pl.BlockSpec((_R, 1), lambda i: (i, 0)),
        ],
        out_shape=[
            jax.ShapeDtypeStruct((N, D), jnp.float32),
            jax.ShapeDtypeStruct((N, 1), jnp.float32),
        ],
    )(h, psum, psum, pcnt, pcnt, Wl, Wr, b.reshape(1, D))


def _tc_layer(h, psum, inv, Wl, Wr, b, relu):
    g = N // _R
    return pl.pallas_call(
        functools.partial(_tc_layer_body, relu),
        grid=(g,),
        in_specs=[
            pl.BlockSpec((_R, D), lambda i: (i, 0)),
            pl.BlockSpec((_R, D), lambda i: (i, 0)),
            pl.BlockSpec((_R, D), lambda i: (i + g, 0)),
            pl.BlockSpec((_R, 1), lambda i: (i, 0)),
            pl.BlockSpec((D, D), lambda i: (0, 0)),
            pl.BlockSpec((D, D), lambda i: (0, 0)),
            pl.BlockSpec((1, D), lambda i: (0, 0)),
        ],
        out_specs=pl.BlockSpec((_R, D), lambda i: (i, 0)),
        out_shape=jax.ShapeDtypeStruct((N, D), jnp.float32),
    )(h, psum, psum, inv, Wl, Wr, b.reshape(1, D))


def kernel(x, edge_index, Wl0, Wr0, b0, Wl1, Wr1, b1, Wl2, Wr2, b2):
    src = edge_index[0].astype(jnp.int32)
    dst = edge_index[1].astype(jnp.int32)
    pad = XROW * C - E
    src3 = jnp.concatenate(
        [src, jnp.zeros((pad,), jnp.int32)]).reshape(XROW, C)
    dst3 = jnp.concatenate(
        [dst, N + (jnp.arange(pad, dtype=jnp.int32) % (NPAD - N))]
    ).reshape(XROW, C)

    (pcnt,) = _sc_cnt(dst3)
    (psum0,) = _sc_agg(x, src3, dst3)
    h1, inv = _tc_layer0(x, psum0, pcnt, Wl0, Wr0, b0)
    (psum1,) = _sc_agg(h1, src3, dst3)
    h2 = _tc_layer(h1, psum1, inv, Wl1, Wr1, b1, True)
    (psum2,) = _sc_agg(h2, src3, dst3)
    return _tc_layer(h2, psum2, inv, Wl2, Wr2, b2, False)


# speedup vs baseline: 3.7496x; 1.0437x over previous
import functools

import jax
import jax.numpy as jnp
from jax import lax
from jax.experimental import pallas as pl
from jax.experimental.pallas import tpu as pltpu
from jax.experimental.pallas import tpu_sc as plsc

N = 10000
D = 128
E = 320000
NC, NS = 2, 16
NW = NC * NS
C = 80
NCHK0 = 192
NCHK1 = 64
NCHKT = NCHK0 + NCHK1
EP = NS * NCHKT * C
XROW = NS * NCHKT
BW = 8
NPAD = 10080
CR = 80
NZCH = NPAD // CR
NOCH = N // CR
MAXZ = -(-NZCH // NS)
MAXO = -(-NOCH // NS)

_MESH = plsc.VectorSubcoreMesh(
    core_axis_name="c", subcore_axis_name="s", num_cores=NC, num_subcores=NS)


def _sc_agg_body(h_hbm, src_hbm, dst_hbm, psum, sidx, didx, rows0, rows1,
                 rows2, zrow, agg_sh, gsem0, gsem1, gsem2):
    core = lax.axis_index("c")
    sub = lax.axis_index("s")
    row0 = sub * NCHKT + core * NCHK0

    def _zero_zrow(i, _):
        for j in range(D // 16):
            zrow[i, pl.ds(j * 16, 16)] = jnp.zeros((16,), jnp.float32)
        return 0
    lax.fori_loop(0, CR, _zero_zrow, 0)
    for k in range(MAXZ):
        ch = k * NS + sub
        @pl.when(ch < NZCH)
        def _():
            pltpu.sync_copy(zrow, agg_sh.at[pl.ds(ch * CR, CR)])

    nblk = jnp.where(core == 0, NCHK0 // BW, NCHK1 // BW)

    plsc.subcore_barrier()

    def _block(b, _):
        r = row0 + b * BW
        pltpu.sync_copy(src_hbm.at[pl.ds(r, BW)], sidx)
        pltpu.sync_copy(dst_hbm.at[pl.ds(r, BW)], didx)
        bufs = (rows0, rows1, rows2)
        sems = (gsem0, gsem1, gsem2)
        pltpu.async_copy(h_hbm.at[sidx.at[0]], rows0, gsem0)
        pltpu.async_copy(h_hbm.at[sidx.at[1]], rows1, gsem1)
        for j in range(BW):
            rj, sj = bufs[j % 3], sems[j % 3]
            pltpu.make_async_copy(h_hbm.at[sidx.at[j]], rj, sj).wait()
            if j + 2 < BW:
                pltpu.async_copy(
                    h_hbm.at[sidx.at[j + 2]], bufs[(j + 2) % 3],
                    sems[(j + 2) % 3])
            pltpu.sync_copy(rj, agg_sh.at[didx.at[j]], add=True)
        return 0
    lax.fori_loop(0, nblk, _block, 0)

    plsc.subcore_barrier()

    for k in range(MAXO):
        ch = k * NS + sub
        @pl.when(ch < NOCH)
        def _():
            pltpu.sync_copy(agg_sh.at[pl.ds(ch * CR, CR)],
                            psum.at[pl.ds(core * N + ch * CR, CR)])


_sc_agg = pl.kernel(
    _sc_agg_body,
    out_type=[jax.ShapeDtypeStruct((2 * N, D), jnp.float32)],
    mesh=_MESH,
    scratch_types=[
        pltpu.VMEM((BW, C), jnp.int32),
        pltpu.VMEM((BW, C), jnp.int32),
        pltpu.VMEM((C, D), jnp.float32),
        pltpu.VMEM((C, D), jnp.float32),
        pltpu.VMEM((C, D), jnp.float32),
        pltpu.VMEM((CR, D), jnp.float32),
        pltpu.VMEM_SHARED((NPAD, D), jnp.float32),
        pltpu.SemaphoreType.DMA,
        pltpu.SemaphoreType.DMA,
        pltpu.SemaphoreType.DMA,
    ],
)


def _sc_cnt_body(dst_hbm, pcnt, didx, ones, cnt_sh):
    core = lax.axis_index("c")
    sub = lax.axis_index("s")
    row0 = (sub * NC + core) * (XROW // NW)
    nblk = XROW // NW // BW

    def _fill_ones(i, _):
        for j in range(D // 16):
            ones[i, pl.ds(j * 16, 16)] = jnp.ones((16,), jnp.float32)
        return 0
    lax.fori_loop(0, C, _fill_ones, 0)
    def _zero_ones(i, _):
        for j in range(D // 16):
            ones[i, pl.ds(j * 16, 16)] = jnp.zeros((16,), jnp.float32)
        return 0
    lax.fori_loop(0, CR, _zero_ones, 0)
    for k in range(MAXZ):
        ch = k * NS + sub
        @pl.when(ch < NZCH)
        def _():
            pltpu.sync_copy(ones.at[pl.ds(0, CR)], cnt_sh.at[pl.ds(ch * CR, CR)])
    lax.fori_loop(0, C, _fill_ones, 0)

    plsc.subcore_barrier()

    def _block(b, _):
        pltpu.sync_copy(dst_hbm.at[pl.ds(row0 + b * BW, BW)], didx)
        for j in range(BW):
            pltpu.sync_copy(ones, cnt_sh.at[didx.at[j]], add=True)
        return 0
    lax.fori_loop(0, nblk, _block, 0)

    plsc.subcore_barrier()

    for k in range(MAXO):
        ch = k * NS + sub
        @pl.when(ch < NOCH)
        def _():
            pltpu.sync_copy(cnt_sh.at[pl.ds(ch * CR, CR)],
                            pcnt.at[pl.ds(core * N + ch * CR, CR)])


_sc_cnt = pl.kernel(
    _sc_cnt_body,
    out_type=[jax.ShapeDtypeStruct((2 * N, D), jnp.float32)],
    mesh=_MESH,
    scratch_types=[
        pltpu.VMEM((BW, C), jnp.int32),
        pltpu.VMEM((C, D), jnp.float32),
        pltpu.VMEM_SHARED((NPAD, D), jnp.float32),
    ],
)


def _tc_layer0_body(hb, p0, p1, c0, c1, wl, wr, bb, ob, invb):
    cnt = c0[:, 0:1] + c1[:, 0:1]
    inv = 1.0 / jnp.maximum(cnt, 1.0)
    agg = (p0[...] + p1[...]) * inv
    out = (jnp.dot(agg, wl[...], preferred_element_type=jnp.float32,
                   precision=lax.Precision.HIGHEST)
           + jnp.dot(hb[...], wr[...], preferred_element_type=jnp.float32,
                     precision=lax.Precision.HIGHEST)
           + bb[...])
    ob[...] = jnp.maximum(out, 0.0)
    invb[...] = inv


def _tc_layer_body(relu, hb, p0, p1, invb, wl, wr, bb, ob):
    agg = (p0[...] + p1[...]) * invb[...]
    out = (jnp.dot(agg, wl[...], preferred_element_type=jnp.float32,
                   precision=lax.Precision.HIGHEST)
           + jnp.dot(hb[...], wr[...], preferred_element_type=jnp.float32,
                     precision=lax.Precision.HIGHEST)
           + bb[...])
    ob[...] = jnp.maximum(out, 0.0) if relu else out


_R = 1000


def _tc_layer0(h, psum, pcnt, Wl, Wr, b):
    g = N // _R
    return pl.pallas_call(
        _tc_layer0_body,
        grid=(g,),
        in_specs=[
            pl.BlockSpec((_R, D), lambda i: (i, 0)),
            pl.BlockSpec((_R, D), lambda i: (i, 0)),
            pl.BlockSpec((_R, D), lambda i: (i + g, 0)),
            pl.BlockSpec((_R, D), lambda i: (i, 0)),
            pl.BlockSpec((_R, D), lambda i: (i + g, 0)),
            pl.BlockSpec((D, D), lambda i: (0, 0)),
            pl.BlockSpec((D, D), lambda i: (0, 0)),
            pl.BlockSpec((1, D), lambda i: (0, 0)),
        ],
        out_specs=[
            pl.BlockSpec((_R, D), lambda i: (i, 0)),
            pl.BlockSpec((_R, 1), lambda i: (i, 0)),
        ],
        out_shape=[
            jax.ShapeDtypeStruct((N, D), jnp.float32),
            jax.ShapeDtypeStruct((N, 1), jnp.float32),
        ],
    )(h, psum, psum, pcnt, pcnt, Wl, Wr, b.reshape(1, D))


def _tc_layer(h, psum, inv, Wl, Wr, b, relu):
    g = N // _R
    return pl.pallas_call(
        functools.partial(_tc_layer_body, relu),
        grid=(g,),
        in_specs=[
            pl.BlockSpec((_R, D), lambda i: (i, 0)),
            pl.BlockSpec((_R, D), lambda i: (i, 0)),
            pl.BlockSpec((_R, D), lambda i: (i + g, 0)),
            pl.BlockSpec((_R, 1), lambda i: (i, 0)),
            pl.BlockSpec((D, D), lambda i: (0, 0)),
            pl.BlockSpec((D, D), lambda i: (0, 0)),
            pl.BlockSpec((1, D), lambda i: (0, 0)),
        ],
        out_specs=pl.BlockSpec((_R, D), lambda i: (i, 0)),
        out_shape=jax.ShapeDtypeStruct((N, D), jnp.float32),
    )(h, psum, psum, inv, Wl, Wr, b.reshape(1, D))


def kernel(x, edge_index, Wl0, Wr0, b0, Wl1, Wr1, b1, Wl2, Wr2, b2):
    src = edge_index[0].astype(jnp.int32)
    dst = edge_index[1].astype(jnp.int32)
    pad = XROW * C - E
    src3 = jnp.concatenate(
        [src, jnp.zeros((pad,), jnp.int32)]).reshape(XROW, C)
    dst3 = jnp.concatenate(
        [dst, N + (jnp.arange(pad, dtype=jnp.int32) % (NPAD - N))]
    ).reshape(XROW, C)

    (pcnt,) = _sc_cnt(dst3)
    (psum0,) = _sc_agg(x, src3, dst3)
    h1, inv = _tc_layer0(x, psum0, pcnt, Wl0, Wr0, b0)
    (psum1,) = _sc_agg(h1, src3, dst3)
    h2 = _tc_layer(h1, psum1, inv, Wl1, Wr1, b1, True)
    (psum2,) = _sc_agg(h2, src3, dst3)
    return _tc_layer(h2, psum2, inv, Wl2, Wr2, b2, False)
